# Initial kernel scaffold; baseline (speedup 1.0000x reference)
import jax
import jax.numpy as jnp
from jax.experimental import pallas as pl


def kernel(h, edge, edge_num, edge_weight, W, a):
    raise NotImplementedError("write your pallas kernel here")


# speedup vs baseline: 15.4727x; 15.4727x over previous
import functools

import jax
import jax.numpy as jnp
from jax import lax
from jax.experimental import pallas as pl
from jax.experimental.pallas import tpu as pltpu
from jax.experimental.pallas import tpu_sc as plsc

N = 10000
E = 320000
F = 128
ALPHA = 0.01
NC = 2
NS = 16
L = 16
NW = NC * NS
EPT = E // NW
K = 80
CH = EPT // K
GROUPS = K // L
RCHUNK = 640


def _tc_prep(h2, W, amat):
    def body(h_ref, w_ref, a_ref, wh_ref, s_ref):
        wh = lax.dot_general(h_ref[...], w_ref[...], (((1,), (1,)), ((), ())),
                             preferred_element_type=jnp.float32)
        wh_ref[...] = wh
        s_ref[...] = lax.dot_general(a_ref[...], wh, (((1,), (1,)), ((), ())),
                                     preferred_element_type=jnp.float32)

    return pl.pallas_call(
        body,
        out_shape=[jax.ShapeDtypeStruct((N, F), jnp.float32),
                   jax.ShapeDtypeStruct((2, N), jnp.float32)],
    )(h2, W, amat)


def _bcast_lane(v16, k):
    idx = jnp.full((L, 1), k, jnp.int32)
    dn = lax.GatherDimensionNumbers(offset_dims=(), collapsed_slice_dims=(0,),
                                    start_index_map=(0,))
    return lax.gather(v16, idx, dn, (1,),
                      mode=lax.GatherScatterMode.PROMISE_IN_BOUNDS)


def _sc_edge(s12, c3, n3, ew3, edge_num, wh):
    mesh = plsc.VectorSubcoreMesh(core_axis_name="c", subcore_axis_name="s",
                                  num_cores=NC, num_subcores=NS)

    @functools.partial(
        pl.kernel,
        out_type=[jax.ShapeDtypeStruct((NC, N, F), jnp.float32),
                  jax.ShapeDtypeStruct((NC, N), jnp.float32)],
        mesh=mesh,
        compiler_params=pltpu.CompilerParams(needs_layout_passes=False,
                                             use_tc_tiling_on_sc=False),
        scratch_types=[
            pltpu.VMEM((N,), jnp.float32),
            pltpu.VMEM((N,), jnp.float32),
            pltpu.VMEM((2, K), jnp.int32),
            pltpu.VMEM((2, K), jnp.int32),
            pltpu.VMEM((2, K), jnp.float32),
            pltpu.VMEM((2, K), jnp.float32),
            pltpu.VMEM((2, K, F), jnp.float32),
            pltpu.VMEM((640,), jnp.float32),
            pltpu.VMEM((L,), jnp.int32),
            pltpu.VMEM_SHARED((N, F), jnp.float32),
            pltpu.VMEM_SHARED((N,), jnp.float32),
            pltpu.SemaphoreType.DMA((2,)),
            pltpu.SemaphoreType.DMA,
            pltpu.SemaphoreType.DMA,
        ],
    )
    def k(s_hbm, c_hbm, n_hbm, ew_hbm, en_hbm, wh_hbm, hp_hbm, dp_hbm,
          s1_v, s2_v, c_v, n_v, ew_v, xe_v, rows_v, zb_v, en_v,
          hp_sp, den_sp, gsem, ssem, dsem):
        cid = lax.axis_index("c")
        sid = lax.axis_index("s")
        wid = cid * NS + sid
        ebase = wid * EPT

        pltpu.sync_copy(s_hbm.at[0], s1_v)
        pltpu.sync_copy(s_hbm.at[1], s2_v)
        pltpu.sync_copy(en_hbm, en_v)

        z16 = jnp.zeros((L,), jnp.float32)

        def zrow(i, _):
            for q in range(F // L):
                rows_v[0, i, pl.ds(q * L, L)] = z16
            return 0

        lax.fori_loop(0, K, zrow, 0)

        def zzb(i, _):
            zb_v[pl.ds(i * L, L)] = z16
            return 0

        lax.fori_loop(0, 640 // L, zzb, 0)

        rbase = sid * RCHUNK
        nrows = lax.select(sid == NS - 1, N - (NS - 1) * RCHUNK, RCHUNK)
        nz = RCHUNK // K
        for t in range(nz):
            @pl.when(t * K < nrows)
            def _z():
                pltpu.sync_copy(rows_v.at[0], hp_sp.at[pl.ds(rbase + t * K, K)])

        @pl.when(sid == 0)
        def _zden():
            for t in range(N // 640):
                pltpu.sync_copy(zb_v, den_sp.at[pl.ds(t * 640, 640)])
            pltpu.sync_copy(zb_v.at[pl.ds(0, N - (N // 640) * 640)],
                            den_sp.at[pl.ds((N // 640) * 640,
                                            N - (N // 640) * 640)])

        plsc.subcore_barrier()

        env = en_v[...]
        en0 = env[0]
        nch = lax.max(0, lax.min(CH, (en0 - ebase + (K - 1)) // K))
        iot = lax.iota(jnp.int32, L)

        def fetch(j, p):
            pltpu.sync_copy(c_hbm.at[wid].at[j], c_v.at[p])
            pltpu.sync_copy(n_hbm.at[wid].at[j], n_v.at[p])
            pltpu.sync_copy(ew_hbm.at[wid].at[j], ew_v.at[p])
            pltpu.async_copy(wh_hbm.at[n_v.at[p]], rows_v.at[p], gsem.at[p])

        @pl.when(nch > 0)
        def _prime():
            fetch(0, 0)

        def chunk(j, _):
            p = lax.rem(j, 2)

            @pl.when(j + 1 < nch)
            def _next():
                fetch(j + 1, 1 - p)

            pltpu.make_async_copy(wh_hbm.at[n_v.at[p]], rows_v.at[p],
                                  gsem.at[p]).wait()

            xes = []
            for g in range(GROUPS):
                c16 = c_v[p, pl.ds(g * L, L)]
                n16 = n_v[p, pl.ds(g * L, L)]
                ew16 = ew_v[p, pl.ds(g * L, L)]
                s1c = plsc.load_gather(s1_v, [c16])
                s2n = plsc.load_gather(s2_v, [n16])
                t = ew16 * (s1c + s2n)
                t = jnp.where(t >= 0.0, t, ALPHA * t)
                xe = jnp.minimum(jnp.exp(t), jnp.float32(1e6))
                ge = ebase + j * K + g * L + iot
                xe = jnp.where(ge < env, xe, jnp.float32(0.0))
                xe_v[p, pl.ds(g * L, L)] = xe
                xes.append(xe)

            for g in range(GROUPS):
                for kl in range(L):
                    s = xes[g][kl]
                    r = g * L + kl
                    for q in range(F // L):
                        rows_v[p, r, pl.ds(q * L, L)] = (
                            rows_v[p, r, pl.ds(q * L, L)] * s)

            d1 = pltpu.async_copy(rows_v.at[p], hp_sp.at[c_v.at[p]], ssem,
                                  add=True)
            d2 = pltpu.async_copy(xe_v.at[p], den_sp.at[c_v.at[p]], dsem,
                                  add=True)
            d1.wait()
            d2.wait()
            return 0

        lax.fori_loop(0, nch, chunk, 0)
        plsc.subcore_barrier()

        for t in range(nz):
            @pl.when(t * K < nrows)
            def _out():
                pltpu.sync_copy(hp_sp.at[pl.ds(rbase + t * K, K)],
                                hp_hbm.at[cid, pl.ds(rbase + t * K, K)])

        @pl.when(sid == 0)
        def _den_out():
            pltpu.sync_copy(den_sp, dp_hbm.at[cid])

    return k(s12, c3, n3, ew3, edge_num, wh)


def _tc_final(hp, dp):
    def body(hp_ref, dp_ref, o_ref):
        den = dp_ref[0, :] + dp_ref[1, :] + jnp.float32(1e-10)
        num = hp_ref[0] + hp_ref[1]
        o_ref[...] = jnp.maximum(num / den[:, None], 0.0)[None]

    return pl.pallas_call(
        body,
        out_shape=jax.ShapeDtypeStruct((1, N, F), jnp.float32),
    )(hp, dp)


def kernel(h, edge, edge_num, edge_weight, W, a):
    h2 = h[0]
    amat = a.reshape(2, F)
    c3 = edge[0, :, 0].reshape(NW, CH, K)
    n3 = edge[0, :, 1].reshape(NW, CH, K)
    ew3 = edge_weight[0].reshape(NW, CH, K)
    en16 = jnp.broadcast_to(edge_num.astype(jnp.int32), (L,))
    wh, s12 = _tc_prep(h2, W, amat)
    hp, dp = _sc_edge(s12, c3, n3, ew3, en16, wh)
    return _tc_final(hp, dp)


# speedup vs baseline: 15.9692x; 1.0321x over previous
import functools

import jax
import jax.numpy as jnp
from jax import lax
from jax.experimental import pallas as pl
from jax.experimental.pallas import tpu as pltpu
from jax.experimental.pallas import tpu_sc as plsc

N = 10000
E = 320000
F = 128
ALPHA = 0.01
NC = 2
NS = 16
L = 16
NW = NC * NS
EPT = E // NW
K = 80
CH = EPT // K
GROUPS = K // L
RCHUNK = 640


def _tc_prep(h2, W, amat):
    def body(h_ref, w_ref, a_ref, wh_ref, s_ref):
        wh = lax.dot_general(h_ref[...], w_ref[...], (((1,), (1,)), ((), ())),
                             preferred_element_type=jnp.float32)
        wh_ref[...] = wh
        s_ref[...] = lax.dot_general(a_ref[...], wh, (((1,), (1,)), ((), ())),
                                     preferred_element_type=jnp.float32)

    return pl.pallas_call(
        body,
        out_shape=[jax.ShapeDtypeStruct((N, F), jnp.float32),
                   jax.ShapeDtypeStruct((2, N), jnp.float32)],
    )(h2, W, amat)


def _sc_edge(s12, c3, n3, ew3, edge_num, wh):
    mesh = plsc.VectorSubcoreMesh(core_axis_name="c", subcore_axis_name="s",
                                  num_cores=NC, num_subcores=NS)

    @functools.partial(
        pl.kernel,
        out_type=[jax.ShapeDtypeStruct((NC, N, F), jnp.float32),
                  jax.ShapeDtypeStruct((NC, N), jnp.float32)],
        mesh=mesh,
        compiler_params=pltpu.CompilerParams(needs_layout_passes=False,
                                             use_tc_tiling_on_sc=False),
        scratch_types=[
            pltpu.VMEM((N,), jnp.float32),
            pltpu.VMEM((N,), jnp.float32),
            pltpu.VMEM((2, K), jnp.int32),
            pltpu.VMEM((2, K), jnp.int32),
            pltpu.VMEM((2, K), jnp.float32),
            pltpu.VMEM((2, K), jnp.float32),
            pltpu.VMEM((2, K, F), jnp.float32),
            pltpu.VMEM((640,), jnp.float32),
            pltpu.VMEM((L,), jnp.int32),
            pltpu.VMEM_SHARED((N, F), jnp.float32),
            pltpu.VMEM_SHARED((N,), jnp.float32),
            pltpu.SemaphoreType.DMA((2,)),
            pltpu.SemaphoreType.DMA,
            pltpu.SemaphoreType.DMA,
        ],
    )
    def k(s_hbm, c_hbm, n_hbm, ew_hbm, en_hbm, wh_hbm, hp_hbm, dp_hbm,
          s1_v, s2_v, c_v, n_v, ew_v, xe_v, rows_v, zb_v, en_v,
          hp_sp, den_sp, gsem, ssem, dsem):
        cid = lax.axis_index("c")
        sid = lax.axis_index("s")
        wid = cid * NS + sid
        ebase = wid * EPT

        pltpu.sync_copy(s_hbm.at[0], s1_v)
        pltpu.sync_copy(s_hbm.at[1], s2_v)
        pltpu.sync_copy(en_hbm, en_v)

        z16 = jnp.zeros((L,), jnp.float32)

        def zrow(i, _):
            for q in range(F // L):
                rows_v[0, i, pl.ds(q * L, L)] = z16
            return 0

        lax.fori_loop(0, K, zrow, 0)

        def zzb(i, _):
            zb_v[pl.ds(i * L, L)] = z16
            return 0

        lax.fori_loop(0, 640 // L, zzb, 0)

        rbase = sid * RCHUNK
        nrows = lax.select(sid == NS - 1, N - (NS - 1) * RCHUNK, RCHUNK)
        nz = RCHUNK // K
        for t in range(nz):
            @pl.when(t * K < nrows)
            def _z():
                pltpu.sync_copy(rows_v.at[0], hp_sp.at[pl.ds(rbase + t * K, K)])

        @pl.when(sid == 0)
        def _zden():
            for t in range(N // 640):
                pltpu.sync_copy(zb_v, den_sp.at[pl.ds(t * 640, 640)])
            pltpu.sync_copy(zb_v.at[pl.ds(0, N - (N // 640) * 640)],
                            den_sp.at[pl.ds((N // 640) * 640,
                                            N - (N // 640) * 640)])

        plsc.subcore_barrier()

        env = en_v[...]
        en0 = env[0]
        nch = lax.max(0, lax.min(CH, (en0 - ebase + (K - 1)) // K))
        iot = lax.iota(jnp.int32, L)

        def fetch(j, p):
            pltpu.sync_copy(c_hbm.at[wid].at[j], c_v.at[p])
            pltpu.sync_copy(n_hbm.at[wid].at[j], n_v.at[p])
            pltpu.sync_copy(ew_hbm.at[wid].at[j], ew_v.at[p])
            pltpu.async_copy(wh_hbm.at[n_v.at[p]], rows_v.at[p], gsem.at[p])

        @pl.when(nch > 0)
        def _prime():
            fetch(0, 0)

        def chunk(j, _):
            p = lax.rem(j, 2)

            xes = []
            for g in range(GROUPS):
                c16 = c_v[p, pl.ds(g * L, L)]
                n16 = n_v[p, pl.ds(g * L, L)]
                ew16 = ew_v[p, pl.ds(g * L, L)]
                s1c = plsc.load_gather(s1_v, [c16])
                s2n = plsc.load_gather(s2_v, [n16])
                t = ew16 * (s1c + s2n)
                t = jnp.where(t >= 0.0, t, ALPHA * t)
                xe = jnp.minimum(jnp.exp(t), jnp.float32(1e6))
                ge = ebase + j * K + g * L + iot
                xe = jnp.where(ge < env, xe, jnp.float32(0.0))
                xe_v[p, pl.ds(g * L, L)] = xe
                xes.append(xe)

            @pl.when(j > 0)
            def _drain_prev():
                pltpu.make_async_copy(rows_v.at[1 - p],
                                      hp_sp.at[c_v.at[1 - p]], ssem).wait()
                pltpu.make_async_copy(xe_v.at[1 - p],
                                      den_sp.at[c_v.at[1 - p]], dsem).wait()

            @pl.when(j + 1 < nch)
            def _next():
                fetch(j + 1, 1 - p)

            pltpu.make_async_copy(wh_hbm.at[n_v.at[p]], rows_v.at[p],
                                  gsem.at[p]).wait()

            for g in range(GROUPS):
                for kl in range(L):
                    s = xes[g][kl]
                    r = g * L + kl
                    for q in range(F // L):
                        rows_v[p, r, pl.ds(q * L, L)] = (
                            rows_v[p, r, pl.ds(q * L, L)] * s)

            pltpu.async_copy(rows_v.at[p], hp_sp.at[c_v.at[p]], ssem,
                             add=True)
            pltpu.async_copy(xe_v.at[p], den_sp.at[c_v.at[p]], dsem,
                             add=True)
            return 0

        lax.fori_loop(0, nch, chunk, 0)

        @pl.when(nch > 0)
        def _drain_last():
            pf = lax.rem(nch - 1, 2)
            pltpu.make_async_copy(rows_v.at[pf], hp_sp.at[c_v.at[pf]],
                                  ssem).wait()
            pltpu.make_async_copy(xe_v.at[pf], den_sp.at[c_v.at[pf]],
                                  dsem).wait()

        plsc.subcore_barrier()

        for t in range(nz):
            @pl.when(t * K < nrows)
            def _out():
                pltpu.sync_copy(hp_sp.at[pl.ds(rbase + t * K, K)],
                                hp_hbm.at[cid, pl.ds(rbase + t * K, K)])

        @pl.when(sid == 0)
        def _den_out():
            pltpu.sync_copy(den_sp, dp_hbm.at[cid])

    return k(s12, c3, n3, ew3, edge_num, wh)


def _tc_final(hp, dp):
    def body(hp_ref, dp_ref, o_ref):
        den = dp_ref[0, :] + dp_ref[1, :] + jnp.float32(1e-10)
        num = hp_ref[0] + hp_ref[1]
        o_ref[...] = jnp.maximum(num / den[:, None], 0.0)[None]

    return pl.pallas_call(
        body,
        out_shape=jax.ShapeDtypeStruct((1, N, F), jnp.float32),
    )(hp, dp)


def kernel(h, edge, edge_num, edge_weight, W, a):
    h2 = h[0]
    amat = a.reshape(2, F)
    c3 = edge[0, :, 0].reshape(NW, CH, K)
    n3 = edge[0, :, 1].reshape(NW, CH, K)
    ew3 = edge_weight[0].reshape(NW, CH, K)
    en16 = jnp.broadcast_to(edge_num.astype(jnp.int32), (L,))
    wh, s12 = _tc_prep(h2, W, amat)
    hp, dp = _sc_edge(s12, c3, n3, ew3, en16, wh)
    return _tc_final(hp, dp)


# speedup vs baseline: 26.1372x; 1.6367x over previous
import functools

import jax
import jax.numpy as jnp
from jax import lax
from jax.experimental import pallas as pl
from jax.experimental.pallas import tpu as pltpu
from jax.experimental.pallas import tpu_sc as plsc

N = 10000
E = 320000
F = 128
ALPHA = 0.01
NC = 2
NS = 16
L = 16
NW = NC * NS
EPT = E // NW
K = 80
CH = EPT // K
GROUPS = K // L
RCHUNK = 640


def _tc_prep(h2, W, amat):
    def body(h_ref, w_ref, a_ref, wh_ref, s_ref):
        wh = lax.dot_general(h_ref[...], w_ref[...], (((1,), (1,)), ((), ())),
                             preferred_element_type=jnp.float32)
        wh_ref[...] = wh
        s_ref[...] = lax.dot_general(a_ref[...], wh, (((1,), (1,)), ((), ())),
                                     preferred_element_type=jnp.float32)

    return pl.pallas_call(
        body,
        out_shape=[jax.ShapeDtypeStruct((N, F), jnp.float32),
                   jax.ShapeDtypeStruct((2, N), jnp.float32)],
    )(h2, W, amat)


def _sc_edge(s12, c3, n3, ew3, edge_num, wh):
    mesh = plsc.VectorSubcoreMesh(core_axis_name="c", subcore_axis_name="s",
                                  num_cores=NC, num_subcores=NS)

    @functools.partial(
        pl.kernel,
        out_type=[jax.ShapeDtypeStruct((NC, N, F), jnp.float32),
                  jax.ShapeDtypeStruct((NC, N), jnp.float32)],
        mesh=mesh,
        compiler_params=pltpu.CompilerParams(needs_layout_passes=False,
                                             use_tc_tiling_on_sc=False),
        scratch_types=[
            pltpu.VMEM((N,), jnp.float32),
            pltpu.VMEM((N,), jnp.float32),
            pltpu.VMEM((2, K), jnp.int32),
            pltpu.VMEM((2, K), jnp.int32),
            pltpu.VMEM((2, K), jnp.float32),
            pltpu.VMEM((2, K), jnp.float32),
            pltpu.VMEM((2, K, F), jnp.float32),
            pltpu.VMEM((640,), jnp.float32),
            pltpu.VMEM((L,), jnp.int32),
            pltpu.VMEM_SHARED((N, F), jnp.float32),
            pltpu.VMEM_SHARED((N,), jnp.float32),
            pltpu.SemaphoreType.DMA((2,)),
            pltpu.SemaphoreType.DMA,
            pltpu.SemaphoreType.DMA,
        ],
    )
    def k(s_hbm, c_hbm, n_hbm, ew_hbm, en_hbm, wh_hbm, hp_hbm, dp_hbm,
          s1_v, s2_v, c_v, n_v, ew_v, xe_v, rows_v, zb_v, en_v,
          hp_sp, den_sp, gsem, ssem, dsem):
        cid = lax.axis_index("c")
        sid = lax.axis_index("s")
        wid = cid * NS + sid

        pltpu.sync_copy(s_hbm.at[0], s1_v)
        pltpu.sync_copy(s_hbm.at[1], s2_v)
        pltpu.sync_copy(en_hbm, en_v)

        z16 = jnp.zeros((L,), jnp.float32)

        def zrow(i, _):
            for q in range(F // L):
                rows_v[0, i, pl.ds(q * L, L)] = z16
            return 0

        lax.fori_loop(0, K, zrow, 0)

        def zzb(i, _):
            zb_v[pl.ds(i * L, L)] = z16
            return 0

        lax.fori_loop(0, 640 // L, zzb, 0)

        rbase = sid * RCHUNK
        nrows = lax.select(sid == NS - 1, N - (NS - 1) * RCHUNK, RCHUNK)
        nz = RCHUNK // K
        for t in range(nz):
            @pl.when(t * K < nrows)
            def _z():
                pltpu.sync_copy(rows_v.at[0], hp_sp.at[pl.ds(rbase + t * K, K)])

        @pl.when(sid == 0)
        def _zden():
            for t in range(N // 640):
                pltpu.sync_copy(zb_v, den_sp.at[pl.ds(t * 640, 640)])
            pltpu.sync_copy(zb_v.at[pl.ds(0, N - (N // 640) * 640)],
                            den_sp.at[pl.ds((N // 640) * 640,
                                            N - (N // 640) * 640)])

        plsc.subcore_barrier()

        env = en_v[...]
        en0 = env[0]
        mtot = (en0 + K - 1) // K
        nch = lax.max(0, lax.min(CH, (mtot - wid + NW - 1) // NW))
        iot = lax.iota(jnp.int32, L)

        def fetch(j, p):
            pltpu.sync_copy(c_hbm.at[wid].at[j], c_v.at[p])
            pltpu.sync_copy(n_hbm.at[wid].at[j], n_v.at[p])
            pltpu.sync_copy(ew_hbm.at[wid].at[j], ew_v.at[p])
            pltpu.async_copy(wh_hbm.at[n_v.at[p]], rows_v.at[p], gsem.at[p])

        @pl.when(nch > 0)
        def _prime():
            fetch(0, 0)

        def chunk(j, _):
            p = lax.rem(j, 2)

            xes = []
            for g in range(GROUPS):
                c16 = c_v[p, pl.ds(g * L, L)]
                n16 = n_v[p, pl.ds(g * L, L)]
                ew16 = ew_v[p, pl.ds(g * L, L)]
                s1c = plsc.load_gather(s1_v, [c16])
                s2n = plsc.load_gather(s2_v, [n16])
                t = ew16 * (s1c + s2n)
                t = jnp.where(t >= 0.0, t, ALPHA * t)
                xe = jnp.minimum(jnp.exp(t), jnp.float32(1e6))
                ge = (j * NW + wid) * K + g * L + iot
                xe = jnp.where(ge < env, xe, jnp.float32(0.0))
                xe_v[p, pl.ds(g * L, L)] = xe
                xes.append(xe)

            @pl.when(j > 0)
            def _drain_prev():
                pltpu.make_async_copy(rows_v.at[1 - p],
                                      hp_sp.at[c_v.at[1 - p]], ssem).wait()
                pltpu.make_async_copy(xe_v.at[1 - p],
                                      den_sp.at[c_v.at[1 - p]], dsem).wait()

            @pl.when(j + 1 < nch)
            def _next():
                fetch(j + 1, 1 - p)

            pltpu.make_async_copy(wh_hbm.at[n_v.at[p]], rows_v.at[p],
                                  gsem.at[p]).wait()

            for g in range(GROUPS):
                for kl in range(L):
                    s = xes[g][kl]
                    r = g * L + kl
                    for q in range(F // L):
                        rows_v[p, r, pl.ds(q * L, L)] = (
                            rows_v[p, r, pl.ds(q * L, L)] * s)

            pltpu.async_copy(rows_v.at[p], hp_sp.at[c_v.at[p]], ssem,
                             add=True)
            pltpu.async_copy(xe_v.at[p], den_sp.at[c_v.at[p]], dsem,
                             add=True)
            return 0

        lax.fori_loop(0, nch, chunk, 0)

        @pl.when(nch > 0)
        def _drain_last():
            pf = lax.rem(nch - 1, 2)
            pltpu.make_async_copy(rows_v.at[pf], hp_sp.at[c_v.at[pf]],
                                  ssem).wait()
            pltpu.make_async_copy(xe_v.at[pf], den_sp.at[c_v.at[pf]],
                                  dsem).wait()

        plsc.subcore_barrier()

        for t in range(nz):
            @pl.when(t * K < nrows)
            def _out():
                pltpu.sync_copy(hp_sp.at[pl.ds(rbase + t * K, K)],
                                hp_hbm.at[cid, pl.ds(rbase + t * K, K)])

        @pl.when(sid == 0)
        def _den_out():
            pltpu.sync_copy(den_sp, dp_hbm.at[cid])

    return k(s12, c3, n3, ew3, edge_num, wh)


def _tc_final(hp, dp):
    def body(hp_ref, dp_ref, o_ref):
        den = dp_ref[0, :] + dp_ref[1, :] + jnp.float32(1e-10)
        num = hp_ref[0] + hp_ref[1]
        o_ref[...] = jnp.maximum(num / den[:, None], 0.0)[None]

    return pl.pallas_call(
        body,
        out_shape=jax.ShapeDtypeStruct((1, N, F), jnp.float32),
    )(hp, dp)


def kernel(h, edge, edge_num, edge_weight, W, a):
    h2 = h[0]
    amat = a.reshape(2, F)
    def _rr(x):
        return x.reshape(CH, NW, K).transpose(1, 0, 2)

    c3 = _rr(edge[0, :, 0])
    n3 = _rr(edge[0, :, 1])
    ew3 = _rr(edge_weight[0])
    en16 = jnp.broadcast_to(edge_num.astype(jnp.int32), (L,))
    wh, s12 = _tc_prep(h2, W, amat)
    hp, dp = _sc_edge(s12, c3, n3, ew3, en16, wh)
    return _tc_final(hp, dp)
